# baseline (speedup 1.0000x reference)
import functools

import jax
import jax.numpy as jnp
from jax.experimental import pallas as pl

_ANCHOR_W = (10.0, 16.0, 33.0)
_ANCHOR_H = (13.0, 30.0, 23.0)
_GRID = 64
_STRIDE = 8.0
_C = 85


def _decode_body(x_ref, o_ref, *, chunk):
    a = pl.program_id(1)
    k = pl.program_id(2)
    v = x_ref[0, 0]
    e = jnp.exp(v)
    sig = e / (1.0 + e)

    row = jax.lax.broadcasted_iota(jnp.int32, v.shape, 0)
    col = jax.lax.broadcasted_iota(jnp.int32, v.shape, 1) + k * chunk
    gx = (col % _GRID).astype(jnp.float32)
    gy = (col // _GRID).astype(jnp.float32)

    aw = jnp.where(a == 0, _ANCHOR_W[0],
                   jnp.where(a == 1, _ANCHOR_W[1], _ANCHOR_W[2]))
    ah = jnp.where(a == 0, _ANCHOR_H[0],
                   jnp.where(a == 1, _ANCHOR_H[1], _ANCHOR_H[2]))

    out = jnp.where(row == 0, (sig + gx) * _STRIDE,
          jnp.where(row == 1, (sig + gy) * _STRIDE,
          jnp.where(row == 2, e * aw,
          jnp.where(row == 3, e * ah, sig))))
    o_ref[0] = out.T


@jax.jit
def kernel(x):
    B = x.shape[0]
    G = x.shape[2]
    S_total = G * G
    A = 3
    K = 4
    S = S_total // K
    x4 = x.reshape(B, A, _C, S_total)
    out = pl.pallas_call(
        functools.partial(_decode_body, chunk=S),
        grid=(B, A, K),
        in_specs=[pl.BlockSpec((1, 1, _C, S), lambda b, a, k: (b, a, 0, k))],
        out_specs=pl.BlockSpec((1, S, _C), lambda b, a, k: (b, a * K + k, 0)),
        out_shape=jax.ShapeDtypeStruct((B, A * S_total, _C), jnp.float32),
    )(x4)
    return out


# speedup vs baseline: 1.1802x; 1.1802x over previous
import functools

import jax
import jax.numpy as jnp
from jax.experimental import pallas as pl

_ANCHOR_W = (10.0, 16.0, 33.0)
_ANCHOR_H = (13.0, 30.0, 23.0)
_GRID = 64
_STRIDE = 8.0
_C = 85


def _decode_body(x_ref, o_ref, *, chunk):
    a = pl.program_id(1)
    k = pl.program_id(2)
    v = x_ref[0, 0]
    e = jnp.exp(v)
    sig = e / (1.0 + e)

    row = jax.lax.broadcasted_iota(jnp.int32, v.shape, 0)
    col = jax.lax.broadcasted_iota(jnp.int32, v.shape, 1) + k * chunk
    gx = (col % _GRID).astype(jnp.float32)
    gy = (col // _GRID).astype(jnp.float32)

    aw = jnp.where(a == 0, _ANCHOR_W[0],
                   jnp.where(a == 1, _ANCHOR_W[1], _ANCHOR_W[2]))
    ah = jnp.where(a == 0, _ANCHOR_H[0],
                   jnp.where(a == 1, _ANCHOR_H[1], _ANCHOR_H[2]))

    out = jnp.where(row == 0, (sig + gx) * _STRIDE,
          jnp.where(row == 1, (sig + gy) * _STRIDE,
          jnp.where(row == 2, e * aw,
          jnp.where(row == 3, e * ah, sig))))
    o_ref[0] = out.T


@jax.jit
def kernel(x):
    B = x.shape[0]
    G = x.shape[2]
    S_total = G * G
    A = 3
    K = 1
    S = S_total // K
    x4 = x.reshape(B, A, _C, S_total)
    out = pl.pallas_call(
        functools.partial(_decode_body, chunk=S),
        grid=(B, A, K),
        in_specs=[pl.BlockSpec((1, 1, _C, S), lambda b, a, k: (b, a, 0, k))],
        out_specs=pl.BlockSpec((1, S, _C), lambda b, a, k: (b, a * K + k, 0)),
        out_shape=jax.ShapeDtypeStruct((B, A * S_total, _C), jnp.float32),
    )(x4)
    return out
